# initial kernel scaffold (unmeasured)
import jax
import jax.numpy as jnp
from jax import lax
from jax.experimental import pallas as pl
from jax.experimental.pallas import tpu as pltpu

N_DEV = 32


def kernel(x, Wg, Wu, Wd):
    m, _ = x.shape
    d = Wd.shape[1]
    chunk = m // N_DEV

    def body(x_ref, wg_ref, wu_ref, wd_ref, out_ref,
             partial_ref, red_ref, rs_buf,
             send_sem1, recv_sem1, send_sem2, recv_sem2):
        my = lax.axis_index("i")

        xv = x_ref[:, :]
        gate = jnp.dot(xv, wg_ref[:, :], preferred_element_type=jnp.float32)
        up = jnp.dot(xv, wu_ref[:, :], preferred_element_type=jnp.float32)
        hidden = gate * (up * jax.nn.sigmoid(up))
        partial_ref[:, :] = jnp.dot(hidden, wd_ref[:, :],
                                    preferred_element_type=jnp.float32)

        sends1 = []
        for off in range(1, N_DEV):
            tgt = lax.rem(my + off, N_DEV)
            rdma = pltpu.make_async_remote_copy(
                src_ref=partial_ref.at[pl.ds(tgt * chunk, chunk), :],
                dst_ref=rs_buf.at[off],
                send_sem=send_sem1.at[off],
                recv_sem=recv_sem1.at[off],
                device_id=(tgt,),
                device_id_type=pl.DeviceIdType.MESH,
            )
            rdma.start()
            sends1.append(rdma)

        acc = partial_ref[pl.ds(my * chunk, chunk), :]
        for off in range(1, N_DEV):
            sends1[off - 1].wait_recv()
            acc = acc + rs_buf[off]
        red_ref[:, :] = acc
        out_ref[pl.ds(my * chunk, chunk), :] = acc

        sends2 = []
        for off in range(1, N_DEV):
            tgt = lax.rem(my + off, N_DEV)
            rdma = pltpu.make_async_remote_copy(
                src_ref=red_ref,
                dst_ref=out_ref.at[pl.ds(my * chunk, chunk), :],
                send_sem=send_sem2.at[off],
                recv_sem=recv_sem2.at[off],
                device_id=(tgt,),
                device_id_type=pl.DeviceIdType.MESH,
            )
            rdma.start()
            sends2.append(rdma)

        for off in range(1, N_DEV):
            src = lax.rem(my - off + N_DEV, N_DEV)
            recv = pltpu.make_async_remote_copy(
                src_ref=red_ref,
                dst_ref=out_ref.at[pl.ds(src * chunk, chunk), :],
                send_sem=send_sem2.at[off],
                recv_sem=recv_sem2.at[off],
                device_id=(src,),
                device_id_type=pl.DeviceIdType.MESH,
            )
            recv.wait_recv()

        for rdma in sends1:
            rdma.wait_send()
        for rdma in sends2:
            rdma.wait_send()

    return pl.pallas_call(
        body,
        out_shape=jax.ShapeDtypeStruct((m, d), jnp.float32),
        in_specs=[pl.BlockSpec(memory_space=pltpu.VMEM)] * 4,
        out_specs=pl.BlockSpec(memory_space=pltpu.VMEM),
        scratch_shapes=[
            pltpu.VMEM((m, d), jnp.float32),
            pltpu.VMEM((chunk, d), jnp.float32),
            pltpu.VMEM((N_DEV, chunk, d), jnp.float32),
            pltpu.SemaphoreType.DMA((N_DEV,)),
            pltpu.SemaphoreType.DMA((N_DEV,)),
            pltpu.SemaphoreType.DMA((N_DEV,)),
            pltpu.SemaphoreType.DMA((N_DEV,)),
        ],
        compiler_params=pltpu.CompilerParams(collective_id=0),
    )(x, Wg, Wu, Wd)


# baseline (device time: 29775 ns/iter reference)
import jax
import jax.numpy as jnp
from jax import lax
from jax.experimental import pallas as pl
from jax.experimental.pallas import tpu as pltpu

N_DEV = 32


def kernel(x, Wg, Wu, Wd):
    m, _ = x.shape
    d = Wd.shape[1]
    chunk = m // N_DEV

    def body(x_ref, wg_ref, wu_ref, wd_ref, out_ref,
             partial_ref, red_ref, rs_buf,
             send_sem1, recv_sem1, send_sem2, recv_sem2):
        my = lax.axis_index("i")

        xv = x_ref[:, :]
        gate = jnp.dot(xv, wg_ref[:, :], preferred_element_type=jnp.float32)
        up = jnp.dot(xv, wu_ref[:, :], preferred_element_type=jnp.float32)
        hidden = gate * (up * jax.nn.sigmoid(up))
        partial_ref[:, :] = jnp.dot(hidden, wd_ref[:, :],
                                    preferred_element_type=jnp.float32)

        sends1 = []
        for off in range(1, N_DEV):
            tgt = lax.rem(my + off, N_DEV)
            rdma = pltpu.make_async_remote_copy(
                src_ref=partial_ref.at[pl.ds(tgt * chunk, chunk), :],
                dst_ref=rs_buf.at[off],
                send_sem=send_sem1.at[off],
                recv_sem=recv_sem1.at[off],
                device_id=(tgt,),
                device_id_type=pl.DeviceIdType.MESH,
            )
            rdma.start()
            sends1.append(rdma)

        acc = partial_ref[pl.ds(my * chunk, chunk), :]
        for off in range(1, N_DEV):
            sends1[off - 1].wait_recv()
            acc = acc + rs_buf[off]
        red_ref[:, :] = acc
        out_ref[pl.ds(my * chunk, chunk), :] = acc

        sends2 = []
        for off in range(1, N_DEV):
            tgt = lax.rem(my + off, N_DEV)
            rdma = pltpu.make_async_remote_copy(
                src_ref=red_ref,
                dst_ref=out_ref.at[pl.ds(my * chunk, chunk), :],
                send_sem=send_sem2.at[off],
                recv_sem=recv_sem2.at[off],
                device_id=(tgt,),
                device_id_type=pl.DeviceIdType.MESH,
            )
            rdma.start()
            sends2.append(rdma)

        for off in range(1, N_DEV):
            src = lax.rem(my - off + N_DEV, N_DEV)
            recv = pltpu.make_async_remote_copy(
                src_ref=red_ref,
                dst_ref=out_ref.at[pl.ds(src * chunk, chunk), :],
                send_sem=send_sem2.at[off],
                recv_sem=recv_sem2.at[off],
                device_id=(src,),
                device_id_type=pl.DeviceIdType.MESH,
            )
            recv.wait_recv()

        for rdma in sends1:
            rdma.wait_send()
        for rdma in sends2:
            rdma.wait_send()

    return pl.pallas_call(
        body,
        out_shape=jax.ShapeDtypeStruct((m, d), jnp.float32),
        in_specs=[pl.BlockSpec(memory_space=pltpu.VMEM)] * 4,
        out_specs=pl.BlockSpec(memory_space=pltpu.VMEM),
        scratch_shapes=[
            pltpu.VMEM((m, d), jnp.float32),
            pltpu.VMEM((chunk, d), jnp.float32),
            pltpu.VMEM((N_DEV, chunk, d), jnp.float32),
            pltpu.SemaphoreType.DMA((N_DEV,)),
            pltpu.SemaphoreType.DMA((N_DEV,)),
            pltpu.SemaphoreType.DMA((N_DEV,)),
            pltpu.SemaphoreType.DMA((N_DEV,)),
        ],
    )(x, Wg, Wu, Wd)
